# baseline (speedup 1.0000x reference)
import functools

import jax
import jax.numpy as jnp
from jax import lax
from jax.experimental import pallas as pl
from jax.experimental.pallas import tpu as pltpu
from jax.experimental.pallas import tpu_sc as plsc

N_NODES = 50000
N_EDGES = 800000
EMB = 64
MAX_DEPTH = 20
EDGE_IN = 16
EDGE_DIM = 16

NC, NS = 2, 16
NW = NC * NS
CHUNK = 128
CPW = 13
BPW = CHUNK * CPW
N_PAD = NW * BPW


def _nodes_body(tid, aid, did, ttab, atab, dtab, out,
                idx_t, idx_a, idx_d, buf_t, buf_a, buf_d,
                sem_t, sem_a, sem_d):
    c = lax.axis_index("c")
    s = lax.axis_index("s")
    wid = s * NC + c
    base = wid * BPW

    def chunk_body(k, carry):
        off = base + k * CHUNK
        pltpu.sync_copy(tid.at[pl.ds(off, CHUNK)], idx_t)
        pltpu.sync_copy(aid.at[pl.ds(off, CHUNK)], idx_a)
        pltpu.sync_copy(did.at[pl.ds(off, CHUNK)], idx_d)

        def clamp_body(i, carry2):
            sl = pl.ds(i * 16, 16)
            idx_d[sl] = jnp.minimum(idx_d[sl], MAX_DEPTH)
            return carry2

        lax.fori_loop(0, CHUNK // 16, clamp_body, 0)

        ct = pltpu.async_copy(ttab.at[idx_t], buf_t, sem_t)
        ca = pltpu.async_copy(atab.at[idx_a], buf_a, sem_a)
        cd = pltpu.async_copy(dtab.at[idx_d], buf_d, sem_d)
        ct.wait()
        ca.wait()
        cd.wait()

        def add_body(r, carry2):
            for q in range(EMB // 16):
                sl = pl.ds(q * 16, 16)
                buf_t[r, sl] = buf_t[r, sl] + buf_a[r, sl] + buf_d[r, sl]
            return carry2

        lax.fori_loop(0, CHUNK, add_body, 0)
        pltpu.sync_copy(buf_t, out.at[pl.ds(off, CHUNK)])
        return carry

    lax.fori_loop(0, CPW, chunk_body, 0)


def _nodes_sc(tid, aid, did, ttab, atab, dtab):
    mesh = plsc.VectorSubcoreMesh(core_axis_name="c", subcore_axis_name="s")
    return pl.kernel(
        _nodes_body,
        out_type=jax.ShapeDtypeStruct((N_PAD, EMB), jnp.float32),
        mesh=mesh,
        scratch_types=[
            pltpu.VMEM((CHUNK,), jnp.int32),
            pltpu.VMEM((CHUNK,), jnp.int32),
            pltpu.VMEM((CHUNK,), jnp.int32),
            pltpu.VMEM((CHUNK, EMB), jnp.float32),
            pltpu.VMEM((CHUNK, EMB), jnp.float32),
            pltpu.VMEM((CHUNK, EMB), jnp.float32),
            pltpu.SemaphoreType.DMA,
            pltpu.SemaphoreType.DMA,
            pltpu.SemaphoreType.DMA,
        ],
        compiler_params=pltpu.CompilerParams(use_tc_tiling_on_sc=False),
    )(tid, aid, did, ttab, atab, dtab)


EDGE_ROWS = N_EDGES // 8
EDGE_BLK = 2000


def _edge_body(w_ref, x_ref, o_ref):
    w = w_ref[...]
    wt = jnp.tile(w, (8, 8))
    ri = lax.broadcasted_iota(jnp.int32, (128, 128), 0) // EDGE_IN
    ci = lax.broadcasted_iota(jnp.int32, (128, 128), 1) // EDGE_DIM
    wb = jnp.where(ri == ci, wt, 0.0)
    o_ref[...] = jnp.dot(x_ref[...], wb, preferred_element_type=jnp.float32)


def _edges_tc(edges2, W_edge):
    return pl.pallas_call(
        _edge_body,
        grid=(EDGE_ROWS // EDGE_BLK,),
        in_specs=[
            pl.BlockSpec((EDGE_IN, EDGE_DIM), lambda i: (0, 0)),
            pl.BlockSpec((EDGE_BLK, 128), lambda i: (i, 0)),
        ],
        out_specs=pl.BlockSpec((EDGE_BLK, 128), lambda i: (i, 0)),
        out_shape=jax.ShapeDtypeStruct((EDGE_ROWS, 128), jnp.float32),
    )(W_edge, edges2)


def kernel(x, depth, edges, type_encoder, attribute_encoder, depth_encoder, W_edge):
    pad = N_PAD - N_NODES
    tid = jnp.pad(x[:, 0], (0, pad))
    aid = jnp.pad(x[:, 1], (0, pad))
    did = jnp.pad(depth[:, 0], (0, pad))

    nodes_pad = _nodes_sc(tid, aid, did, type_encoder, attribute_encoder,
                          depth_encoder)
    nodes = nodes_pad[:N_NODES]

    edges2 = edges.reshape(EDGE_ROWS, 128)
    edges_out = _edges_tc(edges2, W_edge).reshape(N_EDGES, EDGE_DIM)
    return (nodes, edges_out)
